# baseline (speedup 1.0000x reference)
import jax
import jax.numpy as jnp
from jax import lax
from jax.experimental import pallas as pl
from jax.experimental.pallas import tpu as pltpu
from jax.experimental.pallas import tpu_sc as plsc
from jax._src.pallas import mpmd as _plmpmd

B, C, H, W = 1, 128, 384, 384
N, K = 10000, 32
HW = H * W

NC = 200
NSTEPS = N // NC
TRB = 4608

NCORES, NSUB = 2, 16
NTILES = NCORES * NSUB
RPT = (K * N) // NTILES
GCH = 80
NCH = RPT // GCH

RNG = HW // NTILES
PADC = 512
SINK = HW


def _mesh():
    return plsc.VectorSubcoreMesh(core_axis_name="c", subcore_axis_name="s",
                                  num_cores=NCORES, num_subcores=NSUB)


HB = 16


def _trA_body(x_ref, o_ref):
    for h in range(HB):
        o_ref[h] = x_ref[:, h, :].T


def _transpose_cm_to_rm(ft3):
    return pl.pallas_call(
        _trA_body,
        grid=(H // HB,),
        in_specs=[pl.BlockSpec((C, HB, W), lambda i: (0, i, 0))],
        out_specs=pl.BlockSpec((HB, W, C), lambda i: (i, 0, 0)),
        out_shape=jax.ShapeDtypeStruct((H, W, C), jnp.float32),
    )(ft3)


def _trE_body(x_ref, o_ref):
    x3 = x_ref[...].reshape(HB, W, C)
    for h in range(HB):
        o_ref[:, h, :] = x3[h].T


def _transpose_rm_to_cm(x2):
    return pl.pallas_call(
        _trE_body,
        grid=(H // HB,),
        in_specs=[pl.BlockSpec((HB * W, C), lambda i: (i, 0))],
        out_specs=pl.BlockSpec((C, HB, W), lambda i: (0, i, 0)),
        out_shape=jax.ShapeDtypeStruct((C, H, W), jnp.float32),
    )(x2)


def _gather_body(table_hbm, idx3_hbm, out_hbm, idx_v, rows0, rows1, rows2, rows3,
                 sg0, sg1, sg2, sg3, sw0, sw1, sw2, sw3):
    wid = lax.axis_index("s") * NCORES + lax.axis_index("c")
    base = wid * RPT
    rows = (rows0, rows1, rows2, rows3)
    sg = (sg0, sg1, sg2, sg3)
    sw = (sw0, sw1, sw2, sw3)
    pltpu.sync_copy(idx3_hbm.at[wid], idx_v)
    for b in range(3):
        pltpu.async_copy(table_hbm.at[idx_v.at[b]], rows[b], sg[b])

    def arm(jj, a):
        nb = (a + 3) % 4
        pltpu.make_async_copy(table_hbm.at[idx_v.at[jj]], rows[a], sg[a]).wait()
        pltpu.async_copy(rows[a], out_hbm.at[pl.ds(base + jj * GCH, GCH)], sw[a])

        @pl.when(jj > 0)
        def _():
            pltpu.make_async_copy(
                rows[nb], out_hbm.at[pl.ds(base, GCH)], sw[nb]).wait()

        @pl.when(jj + 3 < NCH)
        def _():
            pltpu.async_copy(table_hbm.at[idx_v.at[jj + 3]], rows[nb], sg[nb])

    def chunk(jj, carry):
        for a in range(4):
            @pl.when(jj % 4 == a)
            def _():
                arm(jj, a)
        return carry

    lax.fori_loop(0, NCH, chunk, 0)
    a = (NCH - 1) % 4
    pltpu.make_async_copy(rows[a], out_hbm.at[pl.ds(base, GCH)], sw[a]).wait()


def _sc_gather(table, idx3):
    return pl.kernel(
        _gather_body,
        out_type=jax.ShapeDtypeStruct((N * K, C), jnp.float32),
        mesh=_mesh(),
        scratch_types=[
            pltpu.VMEM((NCH, GCH), jnp.int32),
            pltpu.VMEM((GCH, C), jnp.float32),
            pltpu.VMEM((GCH, C), jnp.float32),
            pltpu.VMEM((GCH, C), jnp.float32),
            pltpu.VMEM((GCH, C), jnp.float32),
        ] + [pltpu.SemaphoreType.DMA] * 8,
    )(table, idx3)


def _main_body(d_ref, f_ref, w1b_ref, w2t_ref, b2_ref, wct_ref, bc_ref,
               bnw_ref, bnb_ref, y_ref, aff_ref, acc_ref):
    i = pl.program_id(0)
    h1 = jnp.maximum(
        lax.dot_general(d_ref[...], w1b_ref[...], (((0,), (0,)), ((), ())),
                        preferred_element_type=jnp.float32), 0.0)
    wk = jnp.maximum(
        jnp.dot(h1.astype(jnp.bfloat16), w2t_ref[...],
                preferred_element_type=jnp.float32) + b2_ref[...], 0.0)
    prod = wk * f_ref[...]
    red = prod.reshape(NC, K, C).sum(axis=1)
    y = jnp.dot(red, wct_ref[...], preferred_element_type=jnp.float32) + bc_ref[...]

    @pl.when(i == 0)
    def _():
        acc_ref[...] = jnp.zeros_like(acc_ref)

    acc_ref[0:1, :] += jnp.sum(y, axis=0, keepdims=True)
    acc_ref[1:2, :] += jnp.sum(y * y, axis=0, keepdims=True)
    y_ref[...] = y

    @pl.when(i == pl.num_programs(0) - 1)
    def _():
        mean = acc_ref[0:1, :] / N
        var = acc_ref[1:2, :] / N - mean * mean
        scale = bnw_ref[...] * lax.rsqrt(var + 1e-5)
        shift = bnb_ref[...] - mean * scale
        aff_ref[...] = jnp.concatenate([scale, shift], axis=0)


def _tc_main(d4, f, w1b, w2t, b2r, wct, bcr, bnwr, bnbr):
    return pl.pallas_call(
        _main_body,
        grid=(NSTEPS,),
        in_specs=[
            pl.BlockSpec((4, NC * K), lambda i: (0, i)),
            pl.BlockSpec((NC * K, C), lambda i: (i, 0)),
            pl.BlockSpec((4, 64), lambda i: (0, 0)),
            pl.BlockSpec((64, C), lambda i: (0, 0)),
            pl.BlockSpec((1, C), lambda i: (0, 0)),
            pl.BlockSpec((C, C), lambda i: (0, 0)),
            pl.BlockSpec((1, C), lambda i: (0, 0)),
            pl.BlockSpec((1, C), lambda i: (0, 0)),
            pl.BlockSpec((1, C), lambda i: (0, 0)),
        ],
        out_specs=[
            pl.BlockSpec((NC, C), lambda i: (i, 0)),
            pl.BlockSpec((2, C), lambda i: (0, 0)),
        ],
        out_shape=[
            jax.ShapeDtypeStruct((N, C), jnp.float32),
            jax.ShapeDtypeStruct((2, C), jnp.float32),
        ],
        scratch_shapes=[pltpu.VMEM((2, C), jnp.float32)],
    )(d4, f, w1b, w2t, b2r, wct, bcr, bnwr, bnbr)


def _aff_body(y_ref, aff_ref, x_ref):
    y = y_ref[...]
    x_ref[...] = jnp.maximum(y * aff_ref[0:1, :] + aff_ref[1:2, :], 0.0)


def _tc_affine(y, aff):
    return pl.pallas_call(
        _aff_body,
        grid=(1,),
        in_specs=[
            pl.BlockSpec((N, C), lambda i: (0, 0)),
            pl.BlockSpec((2, C), lambda i: (0, 0)),
        ],
        out_specs=pl.BlockSpec((N, C), lambda i: (0, 0)),
        out_shape=jax.ShapeDtypeStruct((N, C), jnp.float32),
    )(y, aff)


def _scatter_body(x_hbm, pix_hbm, base_hbm, out_hbm,
                  pix_v, last_v, rows_v, srcn_v, rows2_v, srcn2_v, vals_v,
                  sg, ss):
    wid = lax.axis_index("s") * NCORES + lax.axis_index("c")
    lo = wid * RNG
    lidx = jax.lax.broadcasted_iota(jnp.int32, (16,), 0)

    pltpu.sync_copy(pix_hbm, pix_v)

    def fill(ref, n16, val):
        def go(j, c):
            ref[pl.ds(pl.multiple_of(j * 16, 16), 16)] = jnp.full((16,), val, jnp.int32)
            return c
        lax.fori_loop(0, n16, go, 0)

    fill(last_v, RNG // 16, -1)
    fill(rows_v, (PADC + 32) // 16, SINK)
    fill(srcn_v, (PADC + 32) // 16, 0)

    def scan_step(i, c):
        p = pix_v[i]
        n_vec = i * 16 + lidx
        loc = p - lo
        inr = (loc >= 0) & (loc < RNG)
        key = jnp.where(inr, loc * 16384 + n_vec, jnp.int32(0x7FFFFFFF))
        key_s, n_s = plsc.sort_key_val(key, n_vec)
        loc_s = key_s >> 14
        nxt = loc_s.at[jnp.minimum(lidx + 1, 15)].get(mode="promise_in_bounds")
        win = (loc_s != nxt) | (lidx == 15)
        msk = win & (loc_s < RNG)
        plsc.store_scatter(last_v, [jnp.minimum(loc_s, RNG - 1)], n_s, mask=msk)
        return c

    lax.fori_loop(0, N // 16, scan_step, 0)

    def comp_step(j, cnt):
        lv = last_v[pl.ds(pl.multiple_of(j * 16, 16), 16)]
        m = lv >= 0
        rows_abs = lo + j * 16 + lidx
        plsc.store_compressed(rows_v.at[pl.ds(cnt, 16)], rows_abs, mask=m)
        plsc.store_compressed(srcn_v.at[pl.ds(cnt, 16)], lv, mask=m)
        return cnt + jnp.max(plsc.all_reduce_population_count(m))

    lax.fori_loop(0, RNG // 16, comp_step, 0)

    for j in range(PADC // 128):
        for k in range(8):
            s = j * 128 + k * 16
            rows2_v[j, pl.ds(k * 16, 16)] = rows_v[pl.ds(s, 16)]
            srcn2_v[j, pl.ds(k * 16, 16)] = srcn_v[pl.ds(s, 16)]

    nj = PADC // 128
    pltpu.async_copy(x_hbm.at[srcn2_v.at[0]], vals_v.at[0], sg)
    for j in range(nj):
        b = j % 2
        pltpu.make_async_copy(x_hbm.at[srcn2_v.at[j]], vals_v.at[b], sg).wait()
        if j >= 2:
            pltpu.make_async_copy(
                vals_v.at[b], out_hbm.at[rows2_v.at[j - 2]], ss).wait()
        if j + 1 < nj:
            pltpu.async_copy(x_hbm.at[srcn2_v.at[j + 1]], vals_v.at[1 - b], sg)
        pltpu.async_copy(vals_v.at[b], out_hbm.at[rows2_v.at[j]], ss)
    pltpu.make_async_copy(vals_v.at[0], out_hbm.at[rows2_v.at[nj - 2]], ss).wait()
    pltpu.make_async_copy(vals_v.at[1], out_hbm.at[rows2_v.at[nj - 1]], ss).wait()


def _sc_scatter(x, pix2, base):
    fn = _plmpmd._mpmd_map(
        ((_mesh(), _scatter_body),),
        jax.ShapeDtypeStruct((HW + 8, C), jnp.float32),
        input_output_aliases={2: 0},
        compiler_params=pltpu.CompilerParams(needs_layout_passes=False),
        scratch_types=[
            pltpu.VMEM((N // 16, 16), jnp.int32),
            pltpu.VMEM((RNG,), jnp.int32),
            pltpu.VMEM((PADC + 32,), jnp.int32),
            pltpu.VMEM((PADC + 32,), jnp.int32),
            pltpu.VMEM((PADC // 128, 128), jnp.int32),
            pltpu.VMEM((PADC // 128, 128), jnp.int32),
            pltpu.VMEM((2, 128, C), jnp.float32),
            pltpu.SemaphoreType.DMA,
            pltpu.SemaphoreType.DMA,
        ],
    )
    return fn(x, pix2, base)


def kernel(feature_tensor, nn_diff_pts_3d, pixel_idxs, nn_pixel_idxs,
           W1, b1, W2, b2, Wc, bc, bn_w, bn_b):
    table = _transpose_cm_to_rm(feature_tensor[0]).reshape(HW, C)

    nn_pi = nn_pixel_idxs[0]
    idx = (nn_pi[:, :, 1] * W + nn_pi[:, :, 0]).astype(jnp.int32)
    f = _sc_gather(table, idx.reshape(NTILES, NCH, GCH))

    d4 = jnp.concatenate(
        [nn_diff_pts_3d[0].reshape(N * K, 3).T,
         jnp.ones((1, N * K), jnp.float32)], axis=0)
    w1b = jnp.concatenate([W1.T, b1[None, :]], axis=0)
    y, aff = _tc_main(d4, f, w1b, W2.T.astype(jnp.bfloat16), b2[None, :], Wc.T, bc[None, :],
                      bn_w[None, :], bn_b[None, :])

    x = _tc_affine(y, aff)

    pix = (pixel_idxs[0, :, 1] * W + pixel_idxs[0, :, 0]).astype(jnp.int32)
    base = jnp.zeros((HW + 8, C), jnp.float32)
    scat = _sc_scatter(x, pix.reshape(N // 16, 16), base)
    out = _transpose_rm_to_cm(scat)
    return out.reshape(B, C, H, W)


# speedup vs baseline: 1.6869x; 1.6869x over previous
import jax
import jax.numpy as jnp
from jax import lax
from jax.experimental import pallas as pl
from jax.experimental.pallas import tpu as pltpu
from jax.experimental.pallas import tpu_sc as plsc
from jax._src.pallas import mpmd as _plmpmd

B, C, H, W = 1, 128, 384, 384
N, K = 10000, 32
HW = H * W

NC = 200
NSTEPS = N // NC
TRB = 4608

NCORES, NSUB = 2, 16
NTILES = NCORES * NSUB
RPT = (K * N) // NTILES // 2
GCH = 40
NCH = RPT // GCH

SINK = HW


def _mesh():
    return plsc.VectorSubcoreMesh(core_axis_name="c", subcore_axis_name="s",
                                  num_cores=NCORES, num_subcores=NSUB)


HB = 16


def _trA_body(x_ref, o_ref):
    for h in range(HB):
        o_ref[h] = x_ref[:, h, :].T


def _transpose_cm_to_rm(ft3):
    return pl.pallas_call(
        _trA_body,
        grid=(H // HB,),
        in_specs=[pl.BlockSpec((C, HB, W), lambda i: (0, i, 0))],
        out_specs=pl.BlockSpec((HB, W, C), lambda i: (i, 0, 0)),
        out_shape=jax.ShapeDtypeStruct((H, W, C), jnp.float32),
    )(ft3)


def _trE_body(x_ref, o_ref):
    x3 = x_ref[...].reshape(HB, W, C)
    for h in range(HB):
        o_ref[:, h, :] = x3[h].T


def _transpose_rm_to_cm(x2):
    return pl.pallas_call(
        _trE_body,
        grid=(H // HB,),
        in_specs=[pl.BlockSpec((HB * W, C), lambda i: (i, 0))],
        out_specs=pl.BlockSpec((C, HB, W), lambda i: (0, i, 0)),
        out_shape=jax.ShapeDtypeStruct((C, H, W), jnp.float32),
    )(x2)


def _gather_body(table_hbm, idx3_hbm, out_hbm, idx_v, rows0, rows1, rows2, rows3,
                 sg0, sg1, sg2, sg3, sw0, sw1, sw2, sw3):
    wid = lax.axis_index("s") * NCORES + lax.axis_index("c")
    base = wid * RPT
    rows = (rows0, rows1, rows2, rows3)
    sg = (sg0, sg1, sg2, sg3)
    sw = (sw0, sw1, sw2, sw3)
    pltpu.sync_copy(idx3_hbm.at[wid], idx_v)
    for b in range(3):
        pltpu.async_copy(table_hbm.at[idx_v.at[b]], rows[b], sg[b])

    def arm(jj, a):
        nb = (a + 3) % 4
        pltpu.make_async_copy(table_hbm.at[idx_v.at[jj]], rows[a], sg[a]).wait()
        pltpu.async_copy(rows[a], out_hbm.at[pl.ds(base + jj * GCH, GCH)], sw[a])

        @pl.when(jj > 0)
        def _():
            pltpu.make_async_copy(
                rows[nb], out_hbm.at[pl.ds(base, GCH)], sw[nb]).wait()

        @pl.when(jj + 3 < NCH)
        def _():
            pltpu.async_copy(table_hbm.at[idx_v.at[jj + 3]], rows[nb], sg[nb])

    def chunk(jj, carry):
        for a in range(4):
            @pl.when(jj % 4 == a)
            def _():
                arm(jj, a)
        return carry

    lax.fori_loop(0, NCH, chunk, 0)
    a = (NCH - 1) % 4
    pltpu.make_async_copy(rows[a], out_hbm.at[pl.ds(base, GCH)], sw[a]).wait()


def _sc_gather(table, idx3):
    return pl.kernel(
        _gather_body,
        out_type=jax.ShapeDtypeStruct((N * K // 2, C), jnp.float32),
        mesh=_mesh(),
        scratch_types=[
            pltpu.VMEM((NCH, GCH), jnp.int32),
            pltpu.VMEM((GCH, C), jnp.float32),
            pltpu.VMEM((GCH, C), jnp.float32),
            pltpu.VMEM((GCH, C), jnp.float32),
            pltpu.VMEM((GCH, C), jnp.float32),
        ] + [pltpu.SemaphoreType.DMA] * 8,
    )(table, idx3)


def _main_body(d_ref, f_ref, w1b_ref, w2t_ref, b2_ref, wct_ref, bc_ref,
               bnw_ref, bnb_ref, y_ref, aff_ref, acc_ref):
    i = pl.program_id(0)
    h1 = jnp.maximum(
        lax.dot_general(d_ref[...], w1b_ref[...], (((0,), (0,)), ((), ())),
                        preferred_element_type=jnp.float32), 0.0)
    wk = jnp.maximum(
        jnp.dot(h1.astype(jnp.bfloat16), w2t_ref[...],
                preferred_element_type=jnp.float32) + b2_ref[...], 0.0)
    prod = wk * f_ref[...]
    red = prod.reshape(NC, K, C).sum(axis=1)
    y = jnp.dot(red, wct_ref[...], preferred_element_type=jnp.float32) + bc_ref[...]

    @pl.when(i == 0)
    def _():
        acc_ref[...] = jnp.zeros_like(acc_ref)

    acc_ref[0:1, :] += jnp.sum(y, axis=0, keepdims=True)
    acc_ref[1:2, :] += jnp.sum(y * y, axis=0, keepdims=True)
    y_ref[...] = y

    @pl.when(i == pl.num_programs(0) - 1)
    def _():
        aff_ref[...] = acc_ref[...]


def _tc_main(d4, f, w1b, w2t, b2r, wct, bcr, bnwr, bnbr, half):
    hoff = half * (NSTEPS // 2)
    return pl.pallas_call(
        _main_body,
        grid=(NSTEPS // 2,),
        in_specs=[
            pl.BlockSpec((4, NC * K), lambda i, hoff=hoff: (0, i + hoff)),
            pl.BlockSpec((NC * K, C), lambda i: (i, 0)),
            pl.BlockSpec((4, 64), lambda i: (0, 0)),
            pl.BlockSpec((64, C), lambda i: (0, 0)),
            pl.BlockSpec((1, C), lambda i: (0, 0)),
            pl.BlockSpec((C, C), lambda i: (0, 0)),
            pl.BlockSpec((1, C), lambda i: (0, 0)),
            pl.BlockSpec((1, C), lambda i: (0, 0)),
            pl.BlockSpec((1, C), lambda i: (0, 0)),
        ],
        out_specs=[
            pl.BlockSpec((NC, C), lambda i: (i, 0)),
            pl.BlockSpec((2, C), lambda i: (0, 0)),
        ],
        out_shape=[
            jax.ShapeDtypeStruct((N // 2, C), jnp.float32),
            jax.ShapeDtypeStruct((2, C), jnp.float32),
        ],
        scratch_shapes=[pltpu.VMEM((2, C), jnp.float32)],
    )(d4, f, w1b, w2t, b2r, wct, bcr, bnwr, bnbr)


def _aff_body(y0_ref, y1_ref, s0_ref, s1_ref, bnw_ref, bnb_ref, x_ref):
    s = s0_ref[...] + s1_ref[...]
    mean = s[0:1, :] / N
    var = s[1:2, :] / N - mean * mean
    scale = bnw_ref[...] * lax.rsqrt(var + 1e-5)
    shift = bnb_ref[...] - mean * scale
    x_ref[0:N // 2, :] = jnp.maximum(y0_ref[...] * scale + shift, 0.0)
    x_ref[N // 2:, :] = jnp.maximum(y1_ref[...] * scale + shift, 0.0)


def _tc_affine(y0, y1, s0, s1, bnwr, bnbr):
    return pl.pallas_call(
        _aff_body,
        grid=(1,),
        in_specs=[
            pl.BlockSpec((N // 2, C), lambda i: (0, 0)),
            pl.BlockSpec((N // 2, C), lambda i: (0, 0)),
            pl.BlockSpec((2, C), lambda i: (0, 0)),
            pl.BlockSpec((2, C), lambda i: (0, 0)),
            pl.BlockSpec((1, C), lambda i: (0, 0)),
            pl.BlockSpec((1, C), lambda i: (0, 0)),
        ],
        out_specs=pl.BlockSpec((N, C), lambda i: (0, 0)),
        out_shape=jax.ShapeDtypeStruct((N, C), jnp.float32),
    )(y0, y1, s0, s1, bnwr, bnbr)


SCCH = 80
NSCH = N // SCCH


def _scatter_body(x_hbm, idxo_hbm, base_hbm, out_hbm, idx_v, vals_v, sv, ss):
    wid = lax.axis_index("s") * NCORES + lax.axis_index("c")

    for r in range((NSCH + NTILES - 1) // NTILES):
        cid = wid + r * NTILES

        @pl.when(cid < NSCH)
        def _():
            pltpu.sync_copy(idxo_hbm.at[cid], idx_v)
            pltpu.async_copy(
                x_hbm.at[pl.ds(cid * SCCH, SCCH)], vals_v, sv).wait()
            pltpu.async_copy(vals_v, out_hbm.at[idx_v], ss).wait()


def _sc_scatter(x, idxo, base):
    fn = _plmpmd._mpmd_map(
        ((_mesh(), _scatter_body),),
        jax.ShapeDtypeStruct((HW + 8, C), jnp.float32),
        input_output_aliases={2: 0},
        scratch_types=[
            pltpu.VMEM((SCCH,), jnp.int32),
            pltpu.VMEM((SCCH, C), jnp.float32),
            pltpu.SemaphoreType.DMA,
            pltpu.SemaphoreType.DMA,
        ],
    )
    return fn(x, idxo, base)


def kernel(feature_tensor, nn_diff_pts_3d, pixel_idxs, nn_pixel_idxs,
           W1, b1, W2, b2, Wc, bc, bn_w, bn_b):
    table = _transpose_cm_to_rm(feature_tensor[0]).reshape(HW, C)

    nn_pi = nn_pixel_idxs[0]
    idx = (nn_pi[:, :, 1] * W + nn_pi[:, :, 0]).astype(jnp.int32)
    idx4 = idx.reshape(2, NTILES, NCH, GCH)
    f0 = _sc_gather(table, idx4[0])
    f1 = _sc_gather(table, idx4[1])

    d4 = jnp.concatenate(
        [nn_diff_pts_3d[0].reshape(N * K, 3).T,
         jnp.ones((1, N * K), jnp.float32)], axis=0)
    w1b = jnp.concatenate([W1.T, b1[None, :]], axis=0)
    w2tb = W2.T.astype(jnp.bfloat16)
    args = (w1b, w2tb, b2[None, :], Wc.T, bc[None, :], bn_w[None, :], bn_b[None, :])
    y0, s0 = _tc_main(d4, f0, *args, half=0)
    y1, s1 = _tc_main(d4, f1, *args, half=1)

    x = _tc_affine(y0, y1, s0, s1, bn_w[None, :], bn_b[None, :])

    gate = jnp.minimum(jnp.abs(s0[0, 0]), 0.0).astype(jnp.int32)
    pix = (pixel_idxs[0, :, 1] * W + pixel_idxs[0, :, 0]).astype(jnp.int32) + gate
    narange = jnp.arange(N, dtype=jnp.int32)
    order = jnp.zeros((HW,), jnp.int32).at[pix].max(narange)
    idx_out = jnp.where(order[pix] == narange, pix, SINK)
    base = jnp.zeros((HW + 8, C), jnp.float32)
    scat = _sc_scatter(x, idx_out.reshape(NSCH, SCCH), base)
    out = _transpose_rm_to_cm(scat)
    return out.reshape(B, C, H, W)


# speedup vs baseline: 1.7175x; 1.0181x over previous
import jax
import jax.numpy as jnp
from jax import lax
from jax.experimental import pallas as pl
from jax.experimental.pallas import tpu as pltpu
from jax.experimental.pallas import tpu_sc as plsc
from jax._src.pallas import mpmd as _plmpmd

B, C, H, W = 1, 128, 384, 384
N, K = 10000, 32
HW = H * W

NC = 200
NSTEPS = N // NC
TRB = 4608

NCORES, NSUB = 2, 16
NTILES = NCORES * NSUB
RPT = (K * N) // NTILES // 2
GCH = 40
NCH = RPT // GCH

SINK = HW


def _mesh():
    return plsc.VectorSubcoreMesh(core_axis_name="c", subcore_axis_name="s",
                                  num_cores=NCORES, num_subcores=NSUB)


HB = 16


def _trA_body(x_ref, o_ref):
    for h in range(HB):
        o_ref[h] = x_ref[:, h, :].T


def _transpose_cm_to_rm(ft3):
    return pl.pallas_call(
        _trA_body,
        grid=(H // HB,),
        in_specs=[pl.BlockSpec((C, HB, W), lambda i: (0, i, 0))],
        out_specs=pl.BlockSpec((HB, W, C), lambda i: (i, 0, 0)),
        out_shape=jax.ShapeDtypeStruct((H, W, C), jnp.float32),
    )(ft3)


def _trE_body(x_ref, o_ref):
    x3 = x_ref[...].reshape(HB, W, C)
    for h in range(HB):
        o_ref[:, h, :] = x3[h].T


def _transpose_rm_to_cm(x2):
    return pl.pallas_call(
        _trE_body,
        grid=(H // HB,),
        in_specs=[pl.BlockSpec((HB * W, C), lambda i: (i, 0))],
        out_specs=pl.BlockSpec((C, HB, W), lambda i: (0, i, 0)),
        out_shape=jax.ShapeDtypeStruct((C, H, W), jnp.float32),
    )(x2)


def _gather_body(table_hbm, idx3_hbm, out_hbm, idx_v, rows0, rows1, rows2, rows3,
                 sg0, sg1, sg2, sg3, sw0, sw1, sw2, sw3):
    wid = lax.axis_index("s") * NCORES + lax.axis_index("c")
    base = wid * RPT
    rows = (rows0, rows1, rows2, rows3)
    sg = (sg0, sg1, sg2, sg3)
    sw = (sw0, sw1, sw2, sw3)
    pltpu.sync_copy(idx3_hbm.at[wid], idx_v)
    for b in range(3):
        pltpu.async_copy(table_hbm.at[idx_v.at[b]], rows[b], sg[b])

    def arm(jj, a):
        nb = (a + 3) % 4
        pltpu.make_async_copy(table_hbm.at[idx_v.at[jj]], rows[a], sg[a]).wait()
        pltpu.async_copy(rows[a], out_hbm.at[pl.ds(base + jj * GCH, GCH)], sw[a])

        @pl.when(jj > 0)
        def _():
            pltpu.make_async_copy(
                rows[nb], out_hbm.at[pl.ds(base, GCH)], sw[nb]).wait()

        @pl.when(jj + 3 < NCH)
        def _():
            pltpu.async_copy(table_hbm.at[idx_v.at[jj + 3]], rows[nb], sg[nb])

    def chunk(jj, carry):
        for a in range(4):
            @pl.when(jj % 4 == a)
            def _():
                arm(jj, a)
        return carry

    lax.fori_loop(0, NCH, chunk, 0)
    a = (NCH - 1) % 4
    pltpu.make_async_copy(rows[a], out_hbm.at[pl.ds(base, GCH)], sw[a]).wait()


def _sc_gather(table, idx3):
    return pl.kernel(
        _gather_body,
        out_type=jax.ShapeDtypeStruct((N * K // 2, C), jnp.float32),
        mesh=_mesh(),
        scratch_types=[
            pltpu.VMEM((NCH, GCH), jnp.int32),
            pltpu.VMEM((GCH, C), jnp.float32),
            pltpu.VMEM((GCH, C), jnp.float32),
            pltpu.VMEM((GCH, C), jnp.float32),
            pltpu.VMEM((GCH, C), jnp.float32),
        ] + [pltpu.SemaphoreType.DMA] * 8,
    )(table, idx3)


def _main_body(d_ref, f_ref, w1b_ref, w2t_ref, b2_ref, wct_ref, bc_ref,
               bnw_ref, bnb_ref, y_ref, aff_ref, acc_ref):
    i = pl.program_id(0)
    h1 = jnp.maximum(
        lax.dot_general(d_ref[...], w1b_ref[...], (((0,), (0,)), ((), ())),
                        preferred_element_type=jnp.float32), 0.0)
    wk = jnp.maximum(
        jnp.dot(h1.astype(jnp.bfloat16), w2t_ref[...],
                preferred_element_type=jnp.float32) + b2_ref[...], 0.0)
    prod = wk * f_ref[...]
    red = prod.reshape(NC, K, C).sum(axis=1)
    y = jnp.dot(red, wct_ref[...], preferred_element_type=jnp.float32) + bc_ref[...]

    @pl.when(i == 0)
    def _():
        acc_ref[...] = jnp.zeros_like(acc_ref)

    acc_ref[0:1, :] += jnp.sum(y, axis=0, keepdims=True)
    acc_ref[1:2, :] += jnp.sum(y * y, axis=0, keepdims=True)
    y_ref[...] = y

    @pl.when(i == pl.num_programs(0) - 1)
    def _():
        aff_ref[...] = acc_ref[...]


def _tc_main(d4, f, w1b, w2t, b2r, wct, bcr, bnwr, bnbr, half):
    hoff = half * (NSTEPS // 2)
    return pl.pallas_call(
        _main_body,
        grid=(NSTEPS // 2,),
        in_specs=[
            pl.BlockSpec((4, NC * K), lambda i, hoff=hoff: (0, i + hoff)),
            pl.BlockSpec((NC * K, C), lambda i: (i, 0)),
            pl.BlockSpec((4, 64), lambda i: (0, 0)),
            pl.BlockSpec((64, C), lambda i: (0, 0)),
            pl.BlockSpec((1, C), lambda i: (0, 0)),
            pl.BlockSpec((C, C), lambda i: (0, 0)),
            pl.BlockSpec((1, C), lambda i: (0, 0)),
            pl.BlockSpec((1, C), lambda i: (0, 0)),
            pl.BlockSpec((1, C), lambda i: (0, 0)),
        ],
        out_specs=[
            pl.BlockSpec((NC, C), lambda i: (i, 0)),
            pl.BlockSpec((2, C), lambda i: (0, 0)),
        ],
        out_shape=[
            jax.ShapeDtypeStruct((N // 2, C), jnp.float32),
            jax.ShapeDtypeStruct((2, C), jnp.float32),
        ],
        scratch_shapes=[pltpu.VMEM((2, C), jnp.float32)],
    )(d4, f, w1b, w2t, b2r, wct, bcr, bnwr, bnbr)


def _aff_body(y0_ref, y1_ref, s0_ref, s1_ref, bnw_ref, bnb_ref, x_ref):
    s = s0_ref[...] + s1_ref[...]
    mean = s[0:1, :] / N
    var = s[1:2, :] / N - mean * mean
    scale = bnw_ref[...] * lax.rsqrt(var + 1e-5)
    shift = bnb_ref[...] - mean * scale
    x_ref[0:N // 2, :] = jnp.maximum(y0_ref[...] * scale + shift, 0.0)
    x_ref[N // 2:, :] = jnp.maximum(y1_ref[...] * scale + shift, 0.0)


def _tc_affine(y0, y1, s0, s1, bnwr, bnbr):
    return pl.pallas_call(
        _aff_body,
        grid=(1,),
        in_specs=[
            pl.BlockSpec((N // 2, C), lambda i: (0, 0)),
            pl.BlockSpec((N // 2, C), lambda i: (0, 0)),
            pl.BlockSpec((2, C), lambda i: (0, 0)),
            pl.BlockSpec((2, C), lambda i: (0, 0)),
            pl.BlockSpec((1, C), lambda i: (0, 0)),
            pl.BlockSpec((1, C), lambda i: (0, 0)),
        ],
        out_specs=pl.BlockSpec((N, C), lambda i: (0, 0)),
        out_shape=jax.ShapeDtypeStruct((N, C), jnp.float32),
    )(y0, y1, s0, s1, bnwr, bnbr)


SCCH = 80
NSCH = N // SCCH


def _scatter_body(x_hbm, idxo_hbm, base_hbm, out_hbm, idx_v, vals_v, sv, ss):
    wid = lax.axis_index("s") * NCORES + lax.axis_index("c")

    for r in range((NSCH + NTILES - 1) // NTILES):
        cid = wid + r * NTILES

        @pl.when(cid < NSCH)
        def _():
            pltpu.sync_copy(idxo_hbm.at[cid], idx_v)
            pltpu.async_copy(
                x_hbm.at[pl.ds(cid * SCCH, SCCH)], vals_v, sv).wait()
            pltpu.async_copy(vals_v, out_hbm.at[idx_v], ss).wait()


def _sc_scatter(x, idxo, base):
    fn = _plmpmd._mpmd_map(
        ((_mesh(), _scatter_body),),
        jax.ShapeDtypeStruct((HW + 8, C), jnp.float32),
        input_output_aliases={2: 0},
        scratch_types=[
            pltpu.VMEM((SCCH,), jnp.int32),
            pltpu.VMEM((SCCH, C), jnp.float32),
            pltpu.SemaphoreType.DMA,
            pltpu.SemaphoreType.DMA,
        ],
    )
    return fn(x, idxo, base)


def kernel(feature_tensor, nn_diff_pts_3d, pixel_idxs, nn_pixel_idxs,
           W1, b1, W2, b2, Wc, bc, bn_w, bn_b):
    table = _transpose_cm_to_rm(feature_tensor[0]).reshape(HW, C)

    nn_pi = nn_pixel_idxs[0]
    idx = (nn_pi[:, :, 1] * W + nn_pi[:, :, 0]).astype(jnp.int32)
    idx4 = idx.reshape(2, NTILES, NCH, GCH)
    f0 = _sc_gather(table, idx4[0])
    f1 = _sc_gather(table, idx4[1])

    d4 = jnp.concatenate(
        [nn_diff_pts_3d[0].reshape(N * K, 3).T,
         jnp.ones((1, N * K), jnp.float32)], axis=0)
    w1b = jnp.concatenate([W1.T, b1[None, :]], axis=0)
    w2tb = W2.T.astype(jnp.bfloat16)
    args = (w1b, w2tb, b2[None, :], Wc.T, bc[None, :], bn_w[None, :], bn_b[None, :])
    y0, s0 = _tc_main(d4, f0, *args, half=0)
    y1, s1 = _tc_main(d4, f1, *args, half=1)

    x = _tc_affine(y0, y1, s0, s1, bn_w[None, :], bn_b[None, :])

    gate = jnp.minimum(jnp.abs(s0[0, 0]), 0.0).astype(jnp.int32)
    pix = (pixel_idxs[0, :, 1] * W + pixel_idxs[0, :, 0]).astype(jnp.int32) + gate
    perm = jnp.argsort(pix, stable=True)
    pix_s = pix[perm]
    is_last = jnp.concatenate(
        [pix_s[1:] != pix_s[:-1], jnp.ones((1,), jnp.bool_)])
    idx_out = jnp.zeros((N,), jnp.int32).at[perm].set(
        jnp.where(is_last, pix_s, SINK))
    base = jnp.zeros((HW + 8, C), jnp.float32)
    scat = _sc_scatter(x, idx_out.reshape(NSCH, SCCH), base)
    out = _transpose_rm_to_cm(scat)
    return out.reshape(B, C, H, W)


# speedup vs baseline: 1.7476x; 1.0176x over previous
import functools

import jax
import jax.numpy as jnp
from jax import lax
from jax.experimental import pallas as pl
from jax.experimental.pallas import tpu as pltpu
from jax.experimental.pallas import tpu_sc as plsc
from jax._src.pallas import mpmd as _plmpmd

B, C, H, W = 1, 128, 384, 384
N, K = 10000, 32
HW = H * W

NC = 200
NSTEPS = N // NC
TRB = 4608

NCORES, NSUB = 2, 16
NTILES = NCORES * NSUB
NP1, NP2 = 5200, 4800
GCH = 80

SINK = HW


def _mesh():
    return plsc.VectorSubcoreMesh(core_axis_name="c", subcore_axis_name="s",
                                  num_cores=NCORES, num_subcores=NSUB)


HB = 32


def _trA_body(x_ref, o_ref):
    for h in range(HB):
        o_ref[h] = x_ref[:, h, :].T


def _transpose_cm_to_rm(ft3):
    return pl.pallas_call(
        _trA_body,
        grid=(H // HB,),
        in_specs=[pl.BlockSpec((C, HB, W), lambda i: (0, i, 0))],
        out_specs=pl.BlockSpec((HB, W, C), lambda i: (i, 0, 0)),
        out_shape=jax.ShapeDtypeStruct((H, W, C), jnp.float32),
    )(ft3)


def _trE_body(x_ref, o_ref):
    x3 = x_ref[...].reshape(HB, W, C)
    for h in range(HB):
        o_ref[:, h, :] = x3[h].T


def _transpose_rm_to_cm(x2):
    return pl.pallas_call(
        _trE_body,
        grid=(H // HB,),
        in_specs=[pl.BlockSpec((HB * W, C), lambda i: (i, 0))],
        out_specs=pl.BlockSpec((C, HB, W), lambda i: (0, i, 0)),
        out_shape=jax.ShapeDtypeStruct((C, H, W), jnp.float32),
    )(x2)


def _gather_body(RPT, NCH, table_hbm, idx3_hbm, out_hbm, idx_v, rows0, rows1, rows2, rows3,
                 sg0, sg1, sg2, sg3, sw0, sw1, sw2, sw3):
    wid = lax.axis_index("s") * NCORES + lax.axis_index("c")
    base = wid * RPT
    rows = (rows0, rows1, rows2, rows3)
    sg = (sg0, sg1, sg2, sg3)
    sw = (sw0, sw1, sw2, sw3)
    pltpu.sync_copy(idx3_hbm.at[wid], idx_v)
    for b in range(3):
        pltpu.async_copy(table_hbm.at[idx_v.at[b]], rows[b], sg[b])

    def arm(jj, a):
        nb = (a + 3) % 4
        pltpu.make_async_copy(table_hbm.at[idx_v.at[jj]], rows[a], sg[a]).wait()
        pltpu.async_copy(rows[a], out_hbm.at[pl.ds(base + jj * GCH, GCH)], sw[a])

        @pl.when(jj > 0)
        def _():
            pltpu.make_async_copy(
                rows[nb], out_hbm.at[pl.ds(base, GCH)], sw[nb]).wait()

        @pl.when(jj + 3 < NCH)
        def _():
            pltpu.async_copy(table_hbm.at[idx_v.at[jj + 3]], rows[nb], sg[nb])

    def chunk(jj, carry):
        for a in range(4):
            @pl.when(jj % 4 == a)
            def _():
                arm(jj, a)
        return carry

    lax.fori_loop(0, NCH, chunk, 0)
    a = (NCH - 1) % 4
    pltpu.make_async_copy(rows[a], out_hbm.at[pl.ds(base, GCH)], sw[a]).wait()


def _sc_gather(table, idx3, npts):
    rpt = npts * K // NTILES
    nch = rpt // GCH
    return pl.kernel(
        functools.partial(_gather_body, rpt, nch),
        out_type=jax.ShapeDtypeStruct((npts * K, C), jnp.float32),
        mesh=_mesh(),
        scratch_types=[
            pltpu.VMEM((nch, GCH), jnp.int32),
            pltpu.VMEM((GCH, C), jnp.float32),
            pltpu.VMEM((GCH, C), jnp.float32),
            pltpu.VMEM((GCH, C), jnp.float32),
            pltpu.VMEM((GCH, C), jnp.float32),
        ] + [pltpu.SemaphoreType.DMA] * 8,
    )(table, idx3)


def _main_body(d_ref, f_ref, w1b_ref, w2t_ref, b2_ref, wct_ref, bc_ref,
               bnw_ref, bnb_ref, y_ref, aff_ref, acc_ref):
    i = pl.program_id(0)
    h1 = jnp.maximum(
        lax.dot_general(d_ref[...], w1b_ref[...], (((0,), (0,)), ((), ())),
                        preferred_element_type=jnp.float32), 0.0)
    wk = jnp.maximum(
        jnp.dot(h1.astype(jnp.bfloat16), w2t_ref[...],
                preferred_element_type=jnp.float32) + b2_ref[...], 0.0)
    prod = wk * f_ref[...]
    red = prod.reshape(NC, K, C).sum(axis=1)
    y = jnp.dot(red, wct_ref[...], preferred_element_type=jnp.float32) + bc_ref[...]

    @pl.when(i == 0)
    def _():
        acc_ref[...] = jnp.zeros_like(acc_ref)

    acc_ref[0:1, :] += jnp.sum(y, axis=0, keepdims=True)
    acc_ref[1:2, :] += jnp.sum(y * y, axis=0, keepdims=True)
    y_ref[...] = y

    @pl.when(i == pl.num_programs(0) - 1)
    def _():
        aff_ref[...] = acc_ref[...]


def _tc_main(d4, f, w1b, w2t, b2r, wct, bcr, bnwr, bnbr, npts, step_off):
    return pl.pallas_call(
        _main_body,
        grid=(npts // NC,),
        in_specs=[
            pl.BlockSpec((4, NC * K), lambda i, hoff=step_off: (0, i + hoff)),
            pl.BlockSpec((NC * K, C), lambda i: (i, 0)),
            pl.BlockSpec((4, 64), lambda i: (0, 0)),
            pl.BlockSpec((64, C), lambda i: (0, 0)),
            pl.BlockSpec((1, C), lambda i: (0, 0)),
            pl.BlockSpec((C, C), lambda i: (0, 0)),
            pl.BlockSpec((1, C), lambda i: (0, 0)),
            pl.BlockSpec((1, C), lambda i: (0, 0)),
            pl.BlockSpec((1, C), lambda i: (0, 0)),
        ],
        out_specs=[
            pl.BlockSpec((NC, C), lambda i: (i, 0)),
            pl.BlockSpec((2, C), lambda i: (0, 0)),
        ],
        out_shape=[
            jax.ShapeDtypeStruct((npts, C), jnp.float32),
            jax.ShapeDtypeStruct((2, C), jnp.float32),
        ],
        scratch_shapes=[pltpu.VMEM((2, C), jnp.float32)],
    )(d4, f, w1b, w2t, b2r, wct, bcr, bnwr, bnbr)


def _aff_body(y0_ref, y1_ref, s0_ref, s1_ref, bnw_ref, bnb_ref, x_ref):
    s = s0_ref[...] + s1_ref[...]
    mean = s[0:1, :] / N
    var = s[1:2, :] / N - mean * mean
    scale = bnw_ref[...] * lax.rsqrt(var + 1e-5)
    shift = bnb_ref[...] - mean * scale
    x_ref[0:NP1, :] = jnp.maximum(y0_ref[...] * scale + shift, 0.0)
    x_ref[NP1:, :] = jnp.maximum(y1_ref[...] * scale + shift, 0.0)


def _tc_affine(y0, y1, s0, s1, bnwr, bnbr):
    return pl.pallas_call(
        _aff_body,
        grid=(1,),
        in_specs=[
            pl.BlockSpec((NP1, C), lambda i: (0, 0)),
            pl.BlockSpec((NP2, C), lambda i: (0, 0)),
            pl.BlockSpec((2, C), lambda i: (0, 0)),
            pl.BlockSpec((2, C), lambda i: (0, 0)),
            pl.BlockSpec((1, C), lambda i: (0, 0)),
            pl.BlockSpec((1, C), lambda i: (0, 0)),
        ],
        out_specs=pl.BlockSpec((N, C), lambda i: (0, 0)),
        out_shape=jax.ShapeDtypeStruct((N, C), jnp.float32),
    )(y0, y1, s0, s1, bnwr, bnbr)


SCCH = 80
NSCH = N // SCCH


def _scatter_body(x_hbm, idxo_hbm, base_hbm, out_hbm, idx_v, vals_v, sv, ss):
    wid = lax.axis_index("s") * NCORES + lax.axis_index("c")

    for r in range((NSCH + NTILES - 1) // NTILES):
        cid = wid + r * NTILES

        @pl.when(cid < NSCH)
        def _():
            pltpu.sync_copy(idxo_hbm.at[cid], idx_v)
            pltpu.async_copy(
                x_hbm.at[pl.ds(cid * SCCH, SCCH)], vals_v, sv).wait()
            pltpu.async_copy(vals_v, out_hbm.at[idx_v], ss).wait()


def _sc_scatter(x, idxo, base):
    fn = _plmpmd._mpmd_map(
        ((_mesh(), _scatter_body),),
        jax.ShapeDtypeStruct((HW + 8, C), jnp.float32),
        input_output_aliases={2: 0},
        scratch_types=[
            pltpu.VMEM((SCCH,), jnp.int32),
            pltpu.VMEM((SCCH, C), jnp.float32),
            pltpu.SemaphoreType.DMA,
            pltpu.SemaphoreType.DMA,
        ],
    )
    return fn(x, idxo, base)


def kernel(feature_tensor, nn_diff_pts_3d, pixel_idxs, nn_pixel_idxs,
           W1, b1, W2, b2, Wc, bc, bn_w, bn_b):
    table = _transpose_cm_to_rm(feature_tensor[0]).reshape(HW, C)

    nn_pi = nn_pixel_idxs[0]
    idx = (nn_pi[:, :, 1] * W + nn_pi[:, :, 0]).astype(jnp.int32).reshape(N * K)
    f0 = _sc_gather(table, idx[:NP1 * K].reshape(NTILES, -1, GCH), NP1)
    f1 = _sc_gather(table, idx[NP1 * K:].reshape(NTILES, -1, GCH), NP2)

    d4 = jnp.concatenate(
        [nn_diff_pts_3d[0].reshape(N * K, 3).T,
         jnp.ones((1, N * K), jnp.float32)], axis=0)
    w1b = jnp.concatenate([W1.T, b1[None, :]], axis=0)
    w2tb = W2.T.astype(jnp.bfloat16)
    args = (w1b, w2tb, b2[None, :], Wc.T, bc[None, :], bn_w[None, :], bn_b[None, :])
    y0, s0 = _tc_main(d4, f0, *args, npts=NP1, step_off=0)
    y1, s1 = _tc_main(d4, f1, *args, npts=NP2, step_off=NP1 // NC)

    x = _tc_affine(y0, y1, s0, s1, bn_w[None, :], bn_b[None, :])

    gate = jnp.minimum(jnp.abs(s0[0, 0]), 0.0).astype(jnp.int32)
    pix = (pixel_idxs[0, :, 1] * W + pixel_idxs[0, :, 0]).astype(jnp.int32) + gate
    perm = jnp.argsort(pix, stable=True)
    pix_s = pix[perm]
    is_last = jnp.concatenate(
        [pix_s[1:] != pix_s[:-1], jnp.ones((1,), jnp.bool_)])
    idx_out = jnp.zeros((N,), jnp.int32).at[perm].set(
        jnp.where(is_last, pix_s, SINK))
    base = jnp.zeros((HW + 8, C), jnp.float32)
    scat = _sc_scatter(x, idx_out.reshape(NSCH, SCCH), base)
    out = _transpose_rm_to_cm(scat)
    return out.reshape(B, C, H, W)


# speedup vs baseline: 1.7582x; 1.0061x over previous
import functools

import jax
import jax.numpy as jnp
from jax import lax
from jax.experimental import pallas as pl
from jax.experimental.pallas import tpu as pltpu
from jax.experimental.pallas import tpu_sc as plsc
from jax._src.pallas import mpmd as _plmpmd

B, C, H, W = 1, 128, 384, 384
N, K = 10000, 32
HW = H * W

NC = 200
NSTEPS = N // NC
TRB = 4608

NCORES, NSUB = 2, 16
NTILES = NCORES * NSUB
NP1, NP2 = 5000, 5000
GCH = 40

SINK = HW


def _mesh():
    return plsc.VectorSubcoreMesh(core_axis_name="c", subcore_axis_name="s",
                                  num_cores=NCORES, num_subcores=NSUB)


HB = 32


def _trA_body(x_ref, o_ref):
    for h in range(HB):
        o_ref[h] = x_ref[:, h, :].T


def _transpose_cm_to_rm(ft3):
    return pl.pallas_call(
        _trA_body,
        grid=(H // HB,),
        in_specs=[pl.BlockSpec((C, HB, W), lambda i: (0, i, 0))],
        out_specs=pl.BlockSpec((HB, W, C), lambda i: (i, 0, 0)),
        out_shape=jax.ShapeDtypeStruct((H, W, C), jnp.float32),
    )(ft3)


def _trE_body(x_ref, o_ref):
    x3 = x_ref[...].reshape(HB, W, C)
    for h in range(HB):
        o_ref[:, h, :] = x3[h].T


def _transpose_rm_to_cm(x2):
    return pl.pallas_call(
        _trE_body,
        grid=(H // HB,),
        in_specs=[pl.BlockSpec((HB * W, C), lambda i: (i, 0))],
        out_specs=pl.BlockSpec((C, HB, W), lambda i: (0, i, 0)),
        out_shape=jax.ShapeDtypeStruct((C, H, W), jnp.float32),
    )(x2)


def _gather_body(RPT, NCH, table_hbm, idx3_hbm, out_hbm, idx_v, rows0, rows1, rows2, rows3,
                 sg0, sg1, sg2, sg3, sw0, sw1, sw2, sw3):
    wid = lax.axis_index("s") * NCORES + lax.axis_index("c")
    base = wid * RPT
    rows = (rows0, rows1, rows2, rows3)
    sg = (sg0, sg1, sg2, sg3)
    sw = (sw0, sw1, sw2, sw3)
    pltpu.sync_copy(idx3_hbm.at[wid], idx_v)
    for b in range(3):
        pltpu.async_copy(table_hbm.at[idx_v.at[b]], rows[b], sg[b])

    def arm(jj, a):
        nb = (a + 3) % 4
        pltpu.make_async_copy(table_hbm.at[idx_v.at[jj]], rows[a], sg[a]).wait()
        pltpu.async_copy(rows[a], out_hbm.at[pl.ds(base + jj * GCH, GCH)], sw[a])

        @pl.when(jj > 0)
        def _():
            pltpu.make_async_copy(
                rows[nb], out_hbm.at[pl.ds(base, GCH)], sw[nb]).wait()

        @pl.when(jj + 3 < NCH)
        def _():
            pltpu.async_copy(table_hbm.at[idx_v.at[jj + 3]], rows[nb], sg[nb])

    def chunk(jj, carry):
        for a in range(4):
            @pl.when(jj % 4 == a)
            def _():
                arm(jj, a)
        return carry

    lax.fori_loop(0, NCH, chunk, 0)
    a = (NCH - 1) % 4
    pltpu.make_async_copy(rows[a], out_hbm.at[pl.ds(base, GCH)], sw[a]).wait()


def _sc_gather(table, idx3, npts):
    rpt = npts * K // NTILES
    nch = rpt // GCH
    return pl.kernel(
        functools.partial(_gather_body, rpt, nch),
        out_type=jax.ShapeDtypeStruct((npts * K, C), jnp.float32),
        mesh=_mesh(),
        scratch_types=[
            pltpu.VMEM((nch, GCH), jnp.int32),
            pltpu.VMEM((GCH, C), jnp.float32),
            pltpu.VMEM((GCH, C), jnp.float32),
            pltpu.VMEM((GCH, C), jnp.float32),
            pltpu.VMEM((GCH, C), jnp.float32),
        ] + [pltpu.SemaphoreType.DMA] * 8,
    )(table, idx3)


def _main_body(d_ref, f_ref, w1b_ref, w2t_ref, b2_ref, wct_ref, bc_ref,
               bnw_ref, bnb_ref, y_ref, aff_ref, acc_ref):
    i = pl.program_id(0)
    h1 = jnp.maximum(
        lax.dot_general(d_ref[...], w1b_ref[...], (((0,), (0,)), ((), ())),
                        preferred_element_type=jnp.float32), 0.0)
    wk = jnp.maximum(
        jnp.dot(h1.astype(jnp.bfloat16), w2t_ref[...],
                preferred_element_type=jnp.float32) + b2_ref[...], 0.0)
    prod = wk * f_ref[...]
    red = prod.reshape(NC, K, C).sum(axis=1)
    y = jnp.dot(red, wct_ref[...], preferred_element_type=jnp.float32) + bc_ref[...]

    @pl.when(i == 0)
    def _():
        acc_ref[...] = jnp.zeros_like(acc_ref)

    acc_ref[0:1, :] += jnp.sum(y, axis=0, keepdims=True)
    acc_ref[1:2, :] += jnp.sum(y * y, axis=0, keepdims=True)
    y_ref[...] = y

    @pl.when(i == pl.num_programs(0) - 1)
    def _():
        aff_ref[...] = acc_ref[...]


def _tc_main(d4, f, w1b, w2t, b2r, wct, bcr, bnwr, bnbr, npts, step_off):
    return pl.pallas_call(
        _main_body,
        grid=(npts // NC,),
        in_specs=[
            pl.BlockSpec((4, NC * K), lambda i, hoff=step_off: (0, i + hoff)),
            pl.BlockSpec((NC * K, C), lambda i: (i, 0)),
            pl.BlockSpec((4, 64), lambda i: (0, 0)),
            pl.BlockSpec((64, C), lambda i: (0, 0)),
            pl.BlockSpec((1, C), lambda i: (0, 0)),
            pl.BlockSpec((C, C), lambda i: (0, 0)),
            pl.BlockSpec((1, C), lambda i: (0, 0)),
            pl.BlockSpec((1, C), lambda i: (0, 0)),
            pl.BlockSpec((1, C), lambda i: (0, 0)),
        ],
        out_specs=[
            pl.BlockSpec((NC, C), lambda i: (i, 0)),
            pl.BlockSpec((2, C), lambda i: (0, 0)),
        ],
        out_shape=[
            jax.ShapeDtypeStruct((npts, C), jnp.float32),
            jax.ShapeDtypeStruct((2, C), jnp.float32),
        ],
        scratch_shapes=[pltpu.VMEM((2, C), jnp.float32)],
    )(d4, f, w1b, w2t, b2r, wct, bcr, bnwr, bnbr)


def _aff_body(y0_ref, y1_ref, s0_ref, s1_ref, bnw_ref, bnb_ref, x_ref):
    s = s0_ref[...] + s1_ref[...]
    mean = s[0:1, :] / N
    var = s[1:2, :] / N - mean * mean
    scale = bnw_ref[...] * lax.rsqrt(var + 1e-5)
    shift = bnb_ref[...] - mean * scale
    x_ref[0:NP1, :] = jnp.maximum(y0_ref[...] * scale + shift, 0.0)
    x_ref[NP1:, :] = jnp.maximum(y1_ref[...] * scale + shift, 0.0)


def _tc_affine(y0, y1, s0, s1, bnwr, bnbr):
    return pl.pallas_call(
        _aff_body,
        grid=(1,),
        in_specs=[
            pl.BlockSpec((NP1, C), lambda i: (0, 0)),
            pl.BlockSpec((NP2, C), lambda i: (0, 0)),
            pl.BlockSpec((2, C), lambda i: (0, 0)),
            pl.BlockSpec((2, C), lambda i: (0, 0)),
            pl.BlockSpec((1, C), lambda i: (0, 0)),
            pl.BlockSpec((1, C), lambda i: (0, 0)),
        ],
        out_specs=pl.BlockSpec((N, C), lambda i: (0, 0)),
        out_shape=jax.ShapeDtypeStruct((N, C), jnp.float32),
    )(y0, y1, s0, s1, bnwr, bnbr)


SCCH = 80
NSCH = N // SCCH


def _scatter_body(x_hbm, idxo_hbm, base_hbm, out_hbm, idx_v, vals_v, sv, ss):
    wid = lax.axis_index("s") * NCORES + lax.axis_index("c")

    for r in range((NSCH + NTILES - 1) // NTILES):
        cid = wid + r * NTILES

        @pl.when(cid < NSCH)
        def _():
            pltpu.sync_copy(idxo_hbm.at[cid], idx_v)
            pltpu.async_copy(
                x_hbm.at[pl.ds(cid * SCCH, SCCH)], vals_v, sv).wait()
            pltpu.async_copy(vals_v, out_hbm.at[idx_v], ss).wait()


def _sc_scatter(x, idxo, base):
    fn = _plmpmd._mpmd_map(
        ((_mesh(), _scatter_body),),
        jax.ShapeDtypeStruct((HW + 8, C), jnp.float32),
        input_output_aliases={2: 0},
        scratch_types=[
            pltpu.VMEM((SCCH,), jnp.int32),
            pltpu.VMEM((SCCH, C), jnp.float32),
            pltpu.SemaphoreType.DMA,
            pltpu.SemaphoreType.DMA,
        ],
    )
    return fn(x, idxo, base)


def kernel(feature_tensor, nn_diff_pts_3d, pixel_idxs, nn_pixel_idxs,
           W1, b1, W2, b2, Wc, bc, bn_w, bn_b):
    table = _transpose_cm_to_rm(feature_tensor[0]).reshape(HW, C)

    nn_pi = nn_pixel_idxs[0]
    idx = (nn_pi[:, :, 1] * W + nn_pi[:, :, 0]).astype(jnp.int32).reshape(N * K)
    f0 = _sc_gather(table, idx[:NP1 * K].reshape(NTILES, -1, GCH), NP1)
    f1 = _sc_gather(table, idx[NP1 * K:].reshape(NTILES, -1, GCH), NP2)

    d4 = jnp.concatenate(
        [nn_diff_pts_3d[0].reshape(N * K, 3).T,
         jnp.ones((1, N * K), jnp.float32)], axis=0)
    w1b = jnp.concatenate([W1.T, b1[None, :]], axis=0)
    w2tb = W2.T.astype(jnp.bfloat16)
    args = (w1b, w2tb, b2[None, :], Wc.T, bc[None, :], bn_w[None, :], bn_b[None, :])
    y0, s0 = _tc_main(d4, f0, *args, npts=NP1, step_off=0)
    y1, s1 = _tc_main(d4, f1, *args, npts=NP2, step_off=NP1 // NC)

    x = _tc_affine(y0, y1, s0, s1, bn_w[None, :], bn_b[None, :])

    gate = jnp.minimum(jnp.abs(s0[0, 0]), 0.0).astype(jnp.int32)
    pix = (pixel_idxs[0, :, 1] * W + pixel_idxs[0, :, 0]).astype(jnp.int32) + gate
    perm = jnp.argsort(pix, stable=True)
    pix_s = pix[perm]
    is_last = jnp.concatenate(
        [pix_s[1:] != pix_s[:-1], jnp.ones((1,), jnp.bool_)])
    idx_out = jnp.zeros((N,), jnp.int32).at[perm].set(
        jnp.where(is_last, pix_s, SINK))
    base = jnp.zeros((HW + 8, C), jnp.float32)
    scat = _sc_scatter(x, idx_out.reshape(NSCH, SCCH), base)
    out = _transpose_rm_to_cm(scat)
    return out.reshape(B, C, H, W)
